# baseline (speedup 1.0000x reference)
import functools

import jax
import jax.numpy as jnp
from jax.experimental import pallas as pl
from jax.experimental.pallas import tpu as pltpu
from jax.experimental.pallas import tpu_sc as plsc

D_NODE = 128
LATENT = 128
D_OUT = 16


def _precompute_tables(x, w_rs):
    n = x.shape[0]
    blk = 2000
    nblk = n // blk

    def body(x_ref, w_ref, o_ref):
        o_ref[...] = jnp.dot(x_ref[...], w_ref[...],
                             preferred_element_type=jnp.float32)

    return pl.pallas_call(
        body,
        grid=(2, nblk),
        in_specs=[
            pl.BlockSpec((blk, D_NODE), lambda t, i: (i, 0)),
            pl.BlockSpec((D_NODE, LATENT), lambda t, i: (t, 0)),
        ],
        out_specs=pl.BlockSpec((blk, LATENT), lambda t, i: (t * nblk + i, 0)),
        out_shape=jax.ShapeDtypeStruct((2 * n, LATENT), jnp.float32),
    )(x, w_rs)


def _sc_gather(table, idx):
    b = idx.shape[0]
    v, d = table.shape
    half = v // 2
    e = b // 2
    per_w = e // 16
    win = 160
    mesh = plsc.VectorSubcoreMesh(core_axis_name="core",
                                  subcore_axis_name="subcore")

    @functools.partial(
        pl.kernel,
        out_type=jax.ShapeDtypeStruct((b, d), table.dtype),
        mesh=mesh,
        scratch_types=[
            pltpu.VMEM_SHARED((half, d), table.dtype),
            pltpu.VMEM((win,), jnp.int32),
            pltpu.VMEM((win, d), table.dtype),
            pltpu.SemaphoreType.DMA,
        ],
    )
    def k(t_hbm, i_hbm, o_hbm, shared, idx_v, rows_v, sem):
        cid = jax.lax.axis_index("core")
        sid = jax.lax.axis_index("subcore")

        @pl.when(sid < 5)
        def _():
            pltpu.sync_copy(
                t_hbm.at[pl.ds(cid * half + sid * 2000, 2000)],
                shared.at[pl.ds(sid * 2000, 2000)])

        plsc.subcore_barrier()
        base0 = cid * e + sid * per_w

        @pl.loop(0, per_w, step=win)
        def _(off):
            base = base0 + off
            pltpu.sync_copy(i_hbm.at[pl.ds(base, win)], idx_v)
            pltpu.async_copy(shared.at[idx_v], rows_v, sem).wait()
            pltpu.sync_copy(rows_v, o_hbm.at[pl.ds(base, win)])

    return k(table, idx)


def _mlp_tail(g, ea_t, u, w1e, w1u, b1, w2t, b2_c, gamma_c, beta_c):
    e = ea_t.shape[1]
    blk = 2560
    nblk = e // blk

    def body(gr_ref, gs_ref, ea_ref, u_ref, w1e_ref, w1u_ref, b1_ref,
             w2t_ref, b2_ref, gamma_ref, beta_ref, o_ref):
        h = gr_ref[...] + gs_ref[...]
        h += jax.lax.dot_general(
            ea_ref[...], w1e_ref[...], (((0,), (0,)), ((), ())),
            preferred_element_type=jnp.float32)
        h += jnp.dot(u_ref[...], w1u_ref[...],
                     preferred_element_type=jnp.float32)
        h += b1_ref[...]
        h = jnp.maximum(h, 0.0)
        h2 = jax.lax.dot_general(
            w2t_ref[...], h, (((1,), (1,)), ((), ())),
            preferred_element_type=jnp.float32)
        h2 += b2_ref[...]
        h2 = jnp.maximum(h2, 0.0)
        mean = jnp.mean(h2, axis=0, keepdims=True)
        c = h2 - mean
        var = jnp.mean(c * c, axis=0, keepdims=True)
        o_ref[...] = c / jnp.sqrt(var + 1e-5) * gamma_ref[...] + beta_ref[...]

    return pl.pallas_call(
        body,
        grid=(nblk,),
        in_specs=[
            pl.BlockSpec((blk, LATENT), lambda i: (i, 0)),
            pl.BlockSpec((blk, LATENT), lambda i: (nblk + i, 0)),
            pl.BlockSpec((D_OUT, blk), lambda i: (0, i)),
            pl.BlockSpec((1, D_OUT), lambda i: (0, 0)),
            pl.BlockSpec((D_OUT, LATENT), lambda i: (0, 0)),
            pl.BlockSpec((D_OUT, LATENT), lambda i: (0, 0)),
            pl.BlockSpec((1, LATENT), lambda i: (0, 0)),
            pl.BlockSpec((D_OUT, LATENT), lambda i: (0, 0)),
            pl.BlockSpec((D_OUT, 1), lambda i: (0, 0)),
            pl.BlockSpec((D_OUT, 1), lambda i: (0, 0)),
            pl.BlockSpec((D_OUT, 1), lambda i: (0, 0)),
        ],
        out_specs=pl.BlockSpec((D_OUT, blk), lambda i: (0, i)),
        out_shape=jax.ShapeDtypeStruct((D_OUT, e), jnp.float32),
    )(g, g, ea_t, u, w1e, w1u, b1, w2t, b2_c, gamma_c, beta_c)


def kernel(x, edge_index, edge_attr, u, W1, b1, W2, b2, gamma, beta):
    n = x.shape[0]
    row = edge_index[0].astype(jnp.int32)
    col = edge_index[1].astype(jnp.int32)
    idx = jnp.concatenate([col, row])

    w_rs = W1[: 2 * D_NODE]
    w1e = W1[2 * D_NODE: 2 * D_NODE + D_OUT]
    w1u = W1[2 * D_NODE + D_OUT:]

    table = _precompute_tables(x, w_rs)
    g = _sc_gather(table, idx)
    out_t = _mlp_tail(g, edge_attr.T, u, w1e, w1u,
                      b1.reshape(1, LATENT), W2.T, b2.reshape(D_OUT, 1),
                      gamma.reshape(D_OUT, 1), beta.reshape(D_OUT, 1))
    return out_t.T
